# baseline (speedup 1.0000x reference)
import functools

import jax
import jax.numpy as jnp
from jax import lax
from jax.experimental import pallas as pl
from jax.experimental.pallas import tpu as pltpu
from jax.experimental.pallas import tpu_sc as plsc

N, E, F, D, OUT, G = 10000, 320000, 128, 256, 40, 64
NUM_LAYERS = 4
NT = 16
HALF = D // 2
EPT = E // NT
W = 125
NWIN = EPT // W
CH = 16
NCH = NWIN // CH
RPT = 632
NP = 10240
PT = NP // NT
NB = 2000
GRID = N // NB
f32 = jnp.float32


@functools.lru_cache(maxsize=None)
def _build_deg_kernel():
  mesh = plsc.VectorSubcoreMesh(core_axis_name="c", subcore_axis_name="s")

  @functools.partial(
      pl.kernel, mesh=mesh,
      out_type=jax.ShapeDtypeStruct((NP,), f32),
      scratch_types=[
          pltpu.VMEM((CH, W), jnp.int32),
          pltpu.VMEM((PT,), f32),
          pltpu.VMEM((128,), f32),
          pltpu.VMEM_SHARED((NP,), f32),
      ],
  )
  def _deg_kernel(col3, deg_out, colv, nodebuf, ones, acc):
    c = lax.axis_index("c")
    t = lax.axis_index("s")

    @pl.when(c == 0)
    def _():
        for j in range(128 // 16):
            ones[pl.ds(16 * j, 16)] = jnp.full((16,), 1.0, f32)

        def zb(j, carry):
            nodebuf[pl.ds(16 * j, 16)] = jnp.full((16,), 1.0, f32)
            return carry

        lax.fori_loop(0, PT // 16, zb, 0)
        pbase = pl.multiple_of(t * PT, 8)
        pltpu.sync_copy(nodebuf, acc.at[pl.ds(pbase, PT)])
        plsc.subcore_barrier()

        def chunk(ch, carry):
            pltpu.sync_copy(col3.at[t, pl.ds(ch * CH, CH)], colv)

            def body(w, c2):
                pltpu.sync_copy(ones.at[pl.ds(0, W)], acc.at[colv.at[w]], add=True)
                return c2

            return lax.fori_loop(0, CH, body, carry)

        lax.fori_loop(0, NCH, chunk, 0)
        plsc.subcore_barrier()
        pltpu.sync_copy(acc.at[pl.ds(pbase, PT)], deg_out.at[pl.ds(pbase, PT)])

  return _deg_kernel


def _run_deg(col3):
    return _build_deg_kernel()(col3)


@functools.lru_cache(maxsize=None)
def _build_scatter_kernel():
  mesh = plsc.VectorSubcoreMesh(core_axis_name="c", subcore_axis_name="s")

  @functools.partial(
      pl.kernel, mesh=mesh,
      out_type=(jax.ShapeDtypeStruct((N, HALF), f32),
                jax.ShapeDtypeStruct((N, HALF), f32)),
      scratch_types=[
          pltpu.VMEM((CH, W), jnp.int32),
          pltpu.VMEM((CH, W), jnp.int32),
          pltpu.VMEM((W, HALF), f32),
          pltpu.VMEM((W, HALF), f32),
          pltpu.VMEM_SHARED((N, HALF), f32),
          pltpu.SemaphoreType.DMA,
          pltpu.SemaphoreType.DMA,
      ],
  )
  def _scatter_kernel(hp0, hp1, row3, col3, acc0, acc1, rowv, colv, buf0, buf1,
                      accs, sem0, sem1):
    c = lax.axis_index("c")
    t = lax.axis_index("s")

    def work(hp, acc_out):
        base = pl.multiple_of(t * RPT, 8)

        @pl.when(t < NT - 1)
        def _():
            pltpu.sync_copy(hp.at[pl.ds(base, RPT)], accs.at[pl.ds(base, RPT)])

        @pl.when(t == NT - 1)
        def _():
            last = (NT - 1) * RPT
            pltpu.sync_copy(hp.at[pl.ds(last, N - last)],
                            accs.at[pl.ds(last, N - last)])

        plsc.subcore_barrier()

        def chunk(ch, carry):
            pltpu.sync_copy(row3.at[t, pl.ds(ch * CH, CH)], rowv)
            pltpu.sync_copy(col3.at[t, pl.ds(ch * CH, CH)], colv)
            pltpu.make_async_copy(hp.at[rowv.at[0]], buf0, sem0).start()

            def pair(k, c2):
                w0 = 2 * k
                pltpu.make_async_copy(hp.at[rowv.at[w0 + 1]], buf1, sem1).start()
                pltpu.make_async_copy(hp.at[rowv.at[w0]], buf0, sem0).wait()
                pltpu.sync_copy(buf0, accs.at[colv.at[w0]], add=True)

                @pl.when(k < CH // 2 - 1)
                def _():
                    pltpu.make_async_copy(hp.at[rowv.at[w0 + 2]], buf0,
                                          sem0).start()

                pltpu.make_async_copy(hp.at[rowv.at[w0 + 1]], buf1, sem1).wait()
                pltpu.sync_copy(buf1, accs.at[colv.at[w0 + 1]], add=True)
                return c2

            return lax.fori_loop(0, CH // 2, pair, carry)

        lax.fori_loop(0, NCH, chunk, 0)
        plsc.subcore_barrier()

        @pl.when(t < NT - 1)
        def _():
            pltpu.sync_copy(accs.at[pl.ds(base, RPT)], acc_out.at[pl.ds(base, RPT)])

        @pl.when(t == NT - 1)
        def _():
            last = (NT - 1) * RPT
            pltpu.sync_copy(accs.at[pl.ds(last, N - last)],
                            acc_out.at[pl.ds(last, N - last)])

    @pl.when(c == 0)
    def _():
        work(hp0, acc0)

    @pl.when(c == 1)
    def _():
        work(hp1, acc1)

  return _scatter_kernel


def _run_scatter(hp0, hp1, row3, col3):
    return _build_scatter_kernel()(hp0, hp1, row3, col3)


def _pool_lin(src, batch2, bn, wmat, b):
    din = src.shape[1]
    apply_bn = bn is not None
    next_lin = wmat is not None

    def kfn(*refs):
        it = iter(refs)
        u_ref = next(it)
        if apply_bn:
            st_ref, g_ref, bb_ref = next(it), next(it), next(it)
        bt_ref = next(it)
        if next_lin:
            w_ref, wb_ref = next(it), next(it)
        p_ref = next(it)
        if next_lin:
            t_ref, st1_ref = next(it), next(it)
        i = pl.program_id(0)
        if apply_bn:
            m = st_ref[0:1, :] * (1.0 / N)
            v = st_ref[1:2, :] * (1.0 / N) - m * m
            sc = g_ref[...] * lax.rsqrt(v + 1e-5)
            sh = bb_ref[...] - m * sc
            z = jnp.maximum(u_ref[...] * sc + sh, 0.0)
        else:
            z = u_ref[...]
        seg = bt_ref[...][:, 0]
        oh = (seg[None, :] == lax.broadcasted_iota(jnp.int32, (G, NB), 0)
              ).astype(jnp.bfloat16)
        pb = jnp.dot(oh, z.astype(jnp.bfloat16), preferred_element_type=f32)

        @pl.when(i == 0)
        def _():
            p_ref[...] = pb

        @pl.when(i > 0)
        def _():
            p_ref[...] = p_ref[...] + pb

        if next_lin:
            tv = jnp.dot(z.astype(jnp.bfloat16), w_ref[...].astype(jnp.bfloat16),
                         preferred_element_type=f32) + wb_ref[...]
            t_ref[...] = tv
            s = jnp.sum(tv, axis=0, keepdims=True)
            ss = jnp.sum(tv * tv, axis=0, keepdims=True)
            st1 = jnp.concatenate([s, ss, jnp.zeros((6, D), f32)], axis=0)

            @pl.when(i == 0)
            def _():
                st1_ref[...] = st1

            @pl.when(i > 0)
            def _():
                st1_ref[...] = st1_ref[...] + st1

    in_specs = [pl.BlockSpec((NB, din), lambda i: (i, 0))]
    args = [src]
    if apply_bn:
        in_specs += [pl.BlockSpec((8, din), lambda i: (0, 0)),
                     pl.BlockSpec((1, din), lambda i: (0, 0)),
                     pl.BlockSpec((1, din), lambda i: (0, 0))]
        args += [bn[0], bn[1], bn[2]]
    in_specs += [pl.BlockSpec((NB, 1), lambda i: (i, 0))]
    args += [batch2]
    if next_lin:
        in_specs += [pl.BlockSpec((din, D), lambda i: (0, 0)),
                     pl.BlockSpec((1, D), lambda i: (0, 0))]
        args += [wmat, b]
    out_specs = [pl.BlockSpec((G, din), lambda i: (0, 0))]
    out_shape = [jax.ShapeDtypeStruct((G, din), f32)]
    if next_lin:
        out_specs += [pl.BlockSpec((NB, D), lambda i: (i, 0)),
                      pl.BlockSpec((8, D), lambda i: (0, 0))]
        out_shape += [jax.ShapeDtypeStruct((N, D), f32),
                      jax.ShapeDtypeStruct((8, D), f32)]
    return pl.pallas_call(
        kfn,
        grid=(GRID,),
        in_specs=in_specs,
        out_specs=out_specs,
        out_shape=out_shape,
    )(*args)


def _bn_mm2(tmat, st, g, b, w2, b2, wg, deg):
    def kfn(t_ref, st_ref, g_ref, b_ref, w2_ref, b2_ref, wg_ref, deg_ref,
            hp0_ref, hp1_ref):
        m = st_ref[0:1, :] * (1.0 / N)
        v = st_ref[1:2, :] * (1.0 / N) - m * m
        sc = g_ref[...] * lax.rsqrt(v + 1e-5)
        sh = b_ref[...] - m * sc
        z = jnp.maximum(t_ref[...] * sc + sh, 0.0)
        z2 = jnp.dot(z.astype(jnp.bfloat16), w2_ref[...].astype(jnp.bfloat16),
                     preferred_element_type=f32) + b2_ref[...]
        hp = jnp.dot(z2.astype(jnp.bfloat16), wg_ref[...].astype(jnp.bfloat16),
                     preferred_element_type=f32)
        hp = hp * lax.rsqrt(deg_ref[...])
        hp0_ref[...] = hp[:, :HALF]
        hp1_ref[...] = hp[:, HALF:]

    return pl.pallas_call(
        kfn,
        grid=(GRID,),
        in_specs=[pl.BlockSpec((NB, D), lambda i: (i, 0)),
                  pl.BlockSpec((8, D), lambda i: (0, 0)),
                  pl.BlockSpec((1, D), lambda i: (0, 0)),
                  pl.BlockSpec((1, D), lambda i: (0, 0)),
                  pl.BlockSpec((D, D), lambda i: (0, 0)),
                  pl.BlockSpec((1, D), lambda i: (0, 0)),
                  pl.BlockSpec((D, D), lambda i: (0, 0)),
                  pl.BlockSpec((NB, 1), lambda i: (i, 0))],
        out_specs=[pl.BlockSpec((NB, HALF), lambda i: (i, 0)),
                   pl.BlockSpec((NB, HALF), lambda i: (i, 0))],
        out_shape=[jax.ShapeDtypeStruct((N, HALF), f32),
                   jax.ShapeDtypeStruct((N, HALF), f32)],
    )(tmat, st, g, b, w2, b2, wg, deg)


def _post(acc0, acc1, deg, gb):
    def kfn(a0_ref, a1_ref, deg_ref, gb_ref, u_ref, st_ref):
        i = pl.program_id(0)
        u = jnp.concatenate([a0_ref[...], a1_ref[...]], axis=1)
        u = u * lax.rsqrt(deg_ref[...]) + gb_ref[...]
        u_ref[...] = u
        s = jnp.sum(u, axis=0, keepdims=True)
        ss = jnp.sum(u * u, axis=0, keepdims=True)
        st = jnp.concatenate([s, ss, jnp.zeros((6, D), f32)], axis=0)

        @pl.when(i == 0)
        def _():
            st_ref[...] = st

        @pl.when(i > 0)
        def _():
            st_ref[...] = st_ref[...] + st

    return pl.pallas_call(
        kfn,
        grid=(GRID,),
        in_specs=[pl.BlockSpec((NB, HALF), lambda i: (i, 0)),
                  pl.BlockSpec((NB, HALF), lambda i: (i, 0)),
                  pl.BlockSpec((NB, 1), lambda i: (i, 0)),
                  pl.BlockSpec((1, D), lambda i: (0, 0))],
        out_specs=[pl.BlockSpec((NB, D), lambda i: (i, 0)),
                   pl.BlockSpec((8, D), lambda i: (0, 0))],
        out_shape=[jax.ShapeDtypeStruct((N, D), f32),
                   jax.ShapeDtypeStruct((8, D), f32)],
    )(acc0, acc1, deg, gb)


def _heads(pooled, ws, bs):
    def kfn(*refs):
        o_ref = refs[-1]
        npairs = len(pooled)
        acc = None
        for k in range(npairs):
            p_ref = refs[k]
            w_ref = refs[npairs + k]
            b_ref = refs[2 * npairs + k]
            y = jnp.dot(p_ref[...], w_ref[...], preferred_element_type=f32) + b_ref[...]
            acc = y if acc is None else acc + y
        o_ref[...] = acc

    return pl.pallas_call(
        kfn,
        out_shape=jax.ShapeDtypeStruct((G, OUT), f32),
    )(*pooled, *ws, *bs)


def kernel(x, params, edge_index, batch):
    row3 = edge_index[0].astype(jnp.int32).reshape(NT, NWIN, W)
    col3 = edge_index[1].astype(jnp.int32).reshape(NT, NWIN, W)
    batch2 = batch.astype(jnp.int32).reshape(N, 1)

    deg = _run_deg(col3)[:N].reshape(N, 1)

    pooled = []
    px, tmat, st1 = _pool_lin(x, batch2, None,
                              params["l1W0"], params["l1b0"].reshape(1, D))
    pooled.append(px)
    for i in range(NUM_LAYERS):
        hp0, hp1 = _bn_mm2(tmat, st1,
                           params[f"bn1g{i}"].reshape(1, D),
                           params[f"bn1b{i}"].reshape(1, D),
                           params[f"l2W{i}"], params[f"l2b{i}"].reshape(1, D),
                           params[f"gW{i}"], deg)
        acc0, acc1 = _run_scatter(hp0, hp1, row3, col3)
        u, st2 = _post(acc0, acc1, deg, params[f"gb{i}"].reshape(1, D))
        bn = (st2, params[f"bng{i}"].reshape(1, D), params[f"bnb{i}"].reshape(1, D))
        if i < NUM_LAYERS - 1:
            p, tmat, st1 = _pool_lin(u, batch2, bn,
                                     params[f"l1W{i + 1}"],
                                     params[f"l1b{i + 1}"].reshape(1, D))
        else:
            (p,) = _pool_lin(u, batch2, bn, None, None)
        pooled.append(p)

    return _heads(pooled,
                  [params[f"fcW{i}"] for i in range(NUM_LAYERS + 1)],
                  [params[f"fcb{i}"].reshape(1, OUT) for i in range(NUM_LAYERS + 1)])


# speedup vs baseline: 1.1740x; 1.1740x over previous
import functools

import jax
import jax.numpy as jnp
from jax import lax
from jax.experimental import pallas as pl
from jax.experimental.pallas import tpu as pltpu
from jax.experimental.pallas import tpu_sc as plsc

N, E, F, D, OUT, G = 10000, 320000, 128, 256, 40, 64
NUM_LAYERS = 4
NT = 16
HALF = D // 2
EPT = E // NT
W = 125
NWIN = EPT // W
CH = 16
NCH = NWIN // CH
RPT = 632
NP = 10240
PT = NP // NT
NB = 2000
GRID = N // NB
f32 = jnp.float32


@functools.lru_cache(maxsize=None)
def _build_deg_kernel():
  mesh = plsc.VectorSubcoreMesh(core_axis_name="c", subcore_axis_name="s")

  @functools.partial(
      pl.kernel, mesh=mesh,
      out_type=jax.ShapeDtypeStruct((NP,), f32),
      scratch_types=[
          pltpu.VMEM((CH, W), jnp.int32),
          pltpu.VMEM((PT,), f32),
          pltpu.VMEM((128,), f32),
          pltpu.VMEM_SHARED((NP,), f32),
      ],
  )
  def _deg_kernel(col3, deg_out, colv, nodebuf, ones, acc):
    c = lax.axis_index("c")
    t = lax.axis_index("s")

    @pl.when(c == 0)
    def _():
        for j in range(128 // 16):
            ones[pl.ds(16 * j, 16)] = jnp.full((16,), 1.0, f32)

        def zb(j, carry):
            nodebuf[pl.ds(16 * j, 16)] = jnp.full((16,), 1.0, f32)
            return carry

        lax.fori_loop(0, PT // 16, zb, 0)
        pbase = pl.multiple_of(t * PT, 8)
        pltpu.sync_copy(nodebuf, acc.at[pl.ds(pbase, PT)])
        plsc.subcore_barrier()

        def chunk(ch, carry):
            pltpu.sync_copy(col3.at[t, pl.ds(ch * CH, CH)], colv)

            def body(w, c2):
                pltpu.sync_copy(ones.at[pl.ds(0, W)], acc.at[colv.at[w]], add=True)
                return c2

            return lax.fori_loop(0, CH, body, carry)

        lax.fori_loop(0, NCH, chunk, 0)
        plsc.subcore_barrier()
        pltpu.sync_copy(acc.at[pl.ds(pbase, PT)], deg_out.at[pl.ds(pbase, PT)])

  return _deg_kernel


def _run_deg(col3):
    return _build_deg_kernel()(col3)


@functools.lru_cache(maxsize=None)
def _build_scatter_kernel():
  mesh = plsc.VectorSubcoreMesh(core_axis_name="c", subcore_axis_name="s")

  @functools.partial(
      pl.kernel, mesh=mesh,
      out_type=(jax.ShapeDtypeStruct((N, HALF), f32),
                jax.ShapeDtypeStruct((N, HALF), f32)),
      scratch_types=[
          pltpu.VMEM((2, CH, W), jnp.int32),
          pltpu.VMEM((2, CH, W), jnp.int32),
          pltpu.VMEM((W, HALF), f32),
          pltpu.VMEM((W, HALF), f32),
          pltpu.VMEM_SHARED((N, HALF), f32),
          pltpu.SemaphoreType.DMA,
          pltpu.SemaphoreType.DMA,
          pltpu.SemaphoreType.DMA,
      ],
  )
  def _scatter_kernel(hp0, hp1, row3, col3, acc0, acc1, rowv, colv, buf0, buf1,
                      accs, sem0, sem1, isem):
    c = lax.axis_index("c")
    t = lax.axis_index("s")

    def work(hp, acc_out):
        base = pl.multiple_of(t * RPT, 8)

        @pl.when(t < NT - 1)
        def _():
            pltpu.sync_copy(hp.at[pl.ds(base, RPT)], accs.at[pl.ds(base, RPT)])

        @pl.when(t == NT - 1)
        def _():
            last = (NT - 1) * RPT
            pltpu.sync_copy(hp.at[pl.ds(last, N - last)],
                            accs.at[pl.ds(last, N - last)])

        plsc.subcore_barrier()

        def stage(ch, slot):
            pltpu.make_async_copy(row3.at[t, pl.ds(ch * CH, CH)],
                                  rowv.at[slot], isem).start()
            pltpu.make_async_copy(col3.at[t, pl.ds(ch * CH, CH)],
                                  colv.at[slot], isem).start()

        def stage_wait(ch, slot):
            pltpu.make_async_copy(row3.at[t, pl.ds(ch * CH, CH)],
                                  rowv.at[slot], isem).wait()
            pltpu.make_async_copy(col3.at[t, pl.ds(ch * CH, CH)],
                                  colv.at[slot], isem).wait()

        def process(rv, cv):
            pltpu.make_async_copy(hp.at[rv.at[0]], buf0, sem0).start()

            def pair(k, c2):
                w0 = 2 * k
                pltpu.make_async_copy(hp.at[rv.at[w0 + 1]], buf1, sem1).start()
                pltpu.make_async_copy(hp.at[rv.at[w0]], buf0, sem0).wait()
                pltpu.sync_copy(buf0, accs.at[cv.at[w0]], add=True)

                @pl.when(k < CH // 2 - 1)
                def _():
                    pltpu.make_async_copy(hp.at[rv.at[w0 + 2]], buf0,
                                          sem0).start()

                pltpu.make_async_copy(hp.at[rv.at[w0 + 1]], buf1, sem1).wait()
                pltpu.sync_copy(buf1, accs.at[cv.at[w0 + 1]], add=True)
                return c2

            lax.fori_loop(0, CH // 2, pair, 0)

        stage(0, 0)

        def cpair(p, carry):
            ch0 = 2 * p
            stage_wait(ch0, 0)
            stage(ch0 + 1, 1)
            process(rowv.at[0], colv.at[0])
            stage_wait(ch0 + 1, 1)

            @pl.when(p < NCH // 2 - 1)
            def _():
                stage(ch0 + 2, 0)

            process(rowv.at[1], colv.at[1])
            return carry

        lax.fori_loop(0, NCH // 2, cpair, 0)
        plsc.subcore_barrier()

        @pl.when(t < NT - 1)
        def _():
            pltpu.sync_copy(accs.at[pl.ds(base, RPT)], acc_out.at[pl.ds(base, RPT)])

        @pl.when(t == NT - 1)
        def _():
            last = (NT - 1) * RPT
            pltpu.sync_copy(accs.at[pl.ds(last, N - last)],
                            acc_out.at[pl.ds(last, N - last)])

    @pl.when(c == 0)
    def _():
        work(hp0, acc0)

    @pl.when(c == 1)
    def _():
        work(hp1, acc1)

  return _scatter_kernel


def _run_scatter(hp0, hp1, row3, col3):
    return _build_scatter_kernel()(hp0, hp1, row3, col3)


def _lin_stats(h, wmat, b):
    din = h.shape[1]

    def kfn(h_ref, w_ref, b_ref, t_ref, st_ref):
        i = pl.program_id(0)
        tv = jnp.dot(h_ref[...].astype(jnp.bfloat16),
                     w_ref[...].astype(jnp.bfloat16),
                     preferred_element_type=f32) + b_ref[...]
        t_ref[...] = tv
        s = jnp.sum(tv, axis=0, keepdims=True)
        ss = jnp.sum(tv * tv, axis=0, keepdims=True)
        st = jnp.concatenate([s, ss, jnp.zeros((6, D), f32)], axis=0)

        @pl.when(i == 0)
        def _():
            st_ref[...] = st

        @pl.when(i > 0)
        def _():
            st_ref[...] = st_ref[...] + st

    return pl.pallas_call(
        kfn,
        grid=(GRID,),
        in_specs=[pl.BlockSpec((NB, din), lambda i: (i, 0)),
                  pl.BlockSpec((din, D), lambda i: (0, 0)),
                  pl.BlockSpec((1, D), lambda i: (0, 0))],
        out_specs=[pl.BlockSpec((NB, D), lambda i: (i, 0)),
                   pl.BlockSpec((8, D), lambda i: (0, 0))],
        out_shape=[jax.ShapeDtypeStruct((N, D), f32),
                   jax.ShapeDtypeStruct((8, D), f32)],
    )(h, wmat, b)


def _bn_pool(u, st, g, b, batch2):
    def kfn(u_ref, st_ref, g_ref, b_ref, bt_ref, h_ref, p_ref):
        i = pl.program_id(0)
        m = st_ref[0:1, :] * (1.0 / N)
        v = st_ref[1:2, :] * (1.0 / N) - m * m
        sc = g_ref[...] * lax.rsqrt(v + 1e-5)
        sh = b_ref[...] - m * sc
        z = jnp.maximum(u_ref[...] * sc + sh, 0.0)
        h_ref[...] = z
        seg = bt_ref[...][:, 0]
        oh = (seg[None, :] == lax.broadcasted_iota(jnp.int32, (G, NB), 0)).astype(f32)
        pb = jnp.dot(oh, z, preferred_element_type=f32)

        @pl.when(i == 0)
        def _():
            p_ref[...] = pb

        @pl.when(i > 0)
        def _():
            p_ref[...] = p_ref[...] + pb

    return pl.pallas_call(
        kfn,
        grid=(GRID,),
        in_specs=[pl.BlockSpec((NB, D), lambda i: (i, 0)),
                  pl.BlockSpec((8, D), lambda i: (0, 0)),
                  pl.BlockSpec((1, D), lambda i: (0, 0)),
                  pl.BlockSpec((1, D), lambda i: (0, 0)),
                  pl.BlockSpec((NB, 1), lambda i: (i, 0))],
        out_specs=[pl.BlockSpec((NB, D), lambda i: (i, 0)),
                   pl.BlockSpec((G, D), lambda i: (0, 0))],
        out_shape=[jax.ShapeDtypeStruct((N, D), f32),
                   jax.ShapeDtypeStruct((G, D), f32)],
    )(u, st, g, b, batch2)


def _pool_x(x, batch2):
    def kfn(x_ref, bt_ref, p_ref):
        i = pl.program_id(0)
        seg = bt_ref[...][:, 0]
        oh = (seg[None, :] == lax.broadcasted_iota(jnp.int32, (G, NB), 0)).astype(f32)
        pb = jnp.dot(oh, x_ref[...], preferred_element_type=f32)

        @pl.when(i == 0)
        def _():
            p_ref[...] = pb

        @pl.when(i > 0)
        def _():
            p_ref[...] = p_ref[...] + pb

    return pl.pallas_call(
        kfn,
        grid=(GRID,),
        in_specs=[pl.BlockSpec((NB, F), lambda i: (i, 0)),
                  pl.BlockSpec((NB, 1), lambda i: (i, 0))],
        out_specs=pl.BlockSpec((G, F), lambda i: (0, 0)),
        out_shape=jax.ShapeDtypeStruct((G, F), f32),
    )(x, batch2)


def _bn_mm2(tmat, st, g, b, w2, b2, wg, deg):
    def kfn(t_ref, st_ref, g_ref, b_ref, w2_ref, b2_ref, wg_ref, deg_ref,
            hp0_ref, hp1_ref):
        m = st_ref[0:1, :] * (1.0 / N)
        v = st_ref[1:2, :] * (1.0 / N) - m * m
        sc = g_ref[...] * lax.rsqrt(v + 1e-5)
        sh = b_ref[...] - m * sc
        z = jnp.maximum(t_ref[...] * sc + sh, 0.0)
        z2 = jnp.dot(z.astype(jnp.bfloat16), w2_ref[...].astype(jnp.bfloat16),
                     preferred_element_type=f32) + b2_ref[...]
        hp = jnp.dot(z2.astype(jnp.bfloat16), wg_ref[...].astype(jnp.bfloat16),
                     preferred_element_type=f32)
        hp = hp * lax.rsqrt(deg_ref[...])
        hp0_ref[...] = hp[:, :HALF]
        hp1_ref[...] = hp[:, HALF:]

    return pl.pallas_call(
        kfn,
        grid=(GRID,),
        in_specs=[pl.BlockSpec((NB, D), lambda i: (i, 0)),
                  pl.BlockSpec((8, D), lambda i: (0, 0)),
                  pl.BlockSpec((1, D), lambda i: (0, 0)),
                  pl.BlockSpec((1, D), lambda i: (0, 0)),
                  pl.BlockSpec((D, D), lambda i: (0, 0)),
                  pl.BlockSpec((1, D), lambda i: (0, 0)),
                  pl.BlockSpec((D, D), lambda i: (0, 0)),
                  pl.BlockSpec((NB, 1), lambda i: (i, 0))],
        out_specs=[pl.BlockSpec((NB, HALF), lambda i: (i, 0)),
                   pl.BlockSpec((NB, HALF), lambda i: (i, 0))],
        out_shape=[jax.ShapeDtypeStruct((N, HALF), f32),
                   jax.ShapeDtypeStruct((N, HALF), f32)],
    )(tmat, st, g, b, w2, b2, wg, deg)


def _post(acc0, acc1, deg, gb):
    def kfn(a0_ref, a1_ref, deg_ref, gb_ref, u_ref, st_ref):
        i = pl.program_id(0)
        u = jnp.concatenate([a0_ref[...], a1_ref[...]], axis=1)
        u = u * lax.rsqrt(deg_ref[...]) + gb_ref[...]
        u_ref[...] = u
        s = jnp.sum(u, axis=0, keepdims=True)
        ss = jnp.sum(u * u, axis=0, keepdims=True)
        st = jnp.concatenate([s, ss, jnp.zeros((6, D), f32)], axis=0)

        @pl.when(i == 0)
        def _():
            st_ref[...] = st

        @pl.when(i > 0)
        def _():
            st_ref[...] = st_ref[...] + st

    return pl.pallas_call(
        kfn,
        grid=(GRID,),
        in_specs=[pl.BlockSpec((NB, HALF), lambda i: (i, 0)),
                  pl.BlockSpec((NB, HALF), lambda i: (i, 0)),
                  pl.BlockSpec((NB, 1), lambda i: (i, 0)),
                  pl.BlockSpec((1, D), lambda i: (0, 0))],
        out_specs=[pl.BlockSpec((NB, D), lambda i: (i, 0)),
                   pl.BlockSpec((8, D), lambda i: (0, 0))],
        out_shape=[jax.ShapeDtypeStruct((N, D), f32),
                   jax.ShapeDtypeStruct((8, D), f32)],
    )(acc0, acc1, deg, gb)


def _heads(pooled, ws, bs):
    def kfn(*refs):
        o_ref = refs[-1]
        npairs = len(pooled)
        acc = None
        for k in range(npairs):
            p_ref = refs[k]
            w_ref = refs[npairs + k]
            b_ref = refs[2 * npairs + k]
            y = jnp.dot(p_ref[...], w_ref[...], preferred_element_type=f32) + b_ref[...]
            acc = y if acc is None else acc + y
        o_ref[...] = acc

    return pl.pallas_call(
        kfn,
        out_shape=jax.ShapeDtypeStruct((G, OUT), f32),
    )(*pooled, *ws, *bs)


def kernel(x, params, edge_index, batch):
    row3 = edge_index[0].astype(jnp.int32).reshape(NT, NWIN, W)
    col3 = edge_index[1].astype(jnp.int32).reshape(NT, NWIN, W)
    batch2 = batch.astype(jnp.int32).reshape(N, 1)

    deg = _run_deg(col3)[:N].reshape(N, 1)

    h = x
    pooled = [_pool_x(x, batch2)]
    for i in range(NUM_LAYERS):
        tmat, st1 = _lin_stats(h, params[f"l1W{i}"], params[f"l1b{i}"].reshape(1, D))
        hp0, hp1 = _bn_mm2(tmat, st1,
                           params[f"bn1g{i}"].reshape(1, D),
                           params[f"bn1b{i}"].reshape(1, D),
                           params[f"l2W{i}"], params[f"l2b{i}"].reshape(1, D),
                           params[f"gW{i}"], deg)
        acc0, acc1 = _run_scatter(hp0, hp1, row3, col3)
        u, st2 = _post(acc0, acc1, deg, params[f"gb{i}"].reshape(1, D))
        h, p = _bn_pool(u, st2,
                        params[f"bng{i}"].reshape(1, D),
                        params[f"bnb{i}"].reshape(1, D), batch2)
        pooled.append(p)

    return _heads(pooled,
                  [params[f"fcW{i}"] for i in range(NUM_LAYERS + 1)],
                  [params[f"fcb{i}"].reshape(1, OUT) for i in range(NUM_LAYERS + 1)])
